# baseline (speedup 1.0000x reference)
import jax
import jax.numpy as jnp
from jax.experimental import pallas as pl


def _elementwise_body(logits_ref, boxes_t_ref, count_ref, ts_ref,
                      probs_ref, raw_t_ref, scaled_t_ref, eseq_ref):
    probs_ref[...] = jax.nn.sigmoid(logits_ref[...])

    c = boxes_t_ref[0, 0, :]
    l = boxes_t_ref[0, 1, :]
    lo = c - 0.5 * l
    hi = c + 0.5 * l
    raw_t_ref[0, 0, :] = lo
    raw_t_ref[0, 1, :] = hi
    ts = ts_ref[0, 0, 0]
    scaled_t_ref[0, 0, :] = jnp.clip(lo, 0.0, 1.0) * ts
    scaled_t_ref[0, 1, :] = jnp.clip(hi, 0.0, 1.0) * ts

    cnt = count_ref[0, 0, :]
    m = jnp.max(cnt)
    iota = jax.lax.broadcasted_iota(jnp.int32, cnt.shape, 0)
    first_max = jnp.min(jnp.where(cnt == m, iota, jnp.int32(10**9)))
    eseq_ref[...] = jnp.maximum(first_max, 1).reshape(1, 1, 1)


def kernel(pred_logits, pred_boxes, pred_count, target_sizes):
    N, N_q, N_class = pred_logits.shape
    M = N_q * N_class
    R = M // 128
    logits3d = pred_logits.reshape(N, R, 128)
    boxes_t = pred_boxes.transpose(0, 2, 1)
    count3 = pred_count.reshape(N, 1, pred_count.shape[1])
    ts3 = target_sizes.reshape(N, 1, 1)

    probs, raw_t, scaled_t, eseq3 = pl.pallas_call(
        _elementwise_body,
        grid=(N,),
        in_specs=[
            pl.BlockSpec((1, R, 128), lambda i: (i, 0, 0)),
            pl.BlockSpec((1, 2, N_q), lambda i: (i, 0, 0)),
            pl.BlockSpec((1, 1, pred_count.shape[1]), lambda i: (i, 0, 0)),
            pl.BlockSpec((1, 1, 1), lambda i: (i, 0, 0)),
        ],
        out_specs=[
            pl.BlockSpec((1, R, 128), lambda i: (i, 0, 0)),
            pl.BlockSpec((1, 2, N_q), lambda i: (i, 0, 0)),
            pl.BlockSpec((1, 2, N_q), lambda i: (i, 0, 0)),
            pl.BlockSpec((1, 1, 1), lambda i: (i, 0, 0)),
        ],
        out_shape=[
            jax.ShapeDtypeStruct((N, R, 128), jnp.float32),
            jax.ShapeDtypeStruct((N, 2, N_q), jnp.float32),
            jax.ShapeDtypeStruct((N, 2, N_q), jnp.float32),
            jax.ShapeDtypeStruct((N, 1, 1), jnp.int32),
        ],
    )(logits3d, boxes_t, count3, ts3)

    scores, topk_indexes = jax.lax.top_k(probs.reshape(N, M), N_q)
    topk_boxes = topk_indexes // N_class
    labels = topk_indexes % N_class
    raw_boxes = raw_t.transpose(0, 2, 1)
    scaled = scaled_t.transpose(0, 2, 1)
    gather_idx = jnp.repeat(topk_boxes[..., None], 2, axis=-1)
    boxes = jnp.take_along_axis(scaled, gather_idx, axis=1)
    eseq_lens = eseq3.reshape(N)
    return scores, labels, boxes, raw_boxes, topk_boxes, eseq_lens, topk_indexes


# speedup vs baseline: 6.4977x; 6.4977x over previous
import functools

import jax
import jax.numpy as jnp
from jax import lax
from jax.experimental import pallas as pl
from jax.experimental.pallas import tpu as pltpu
from jax.experimental.pallas import tpu_sc as plsc

N = 16
N_Q = 5000
N_CLASS = 80
M = N_Q * N_CLASS
HALF = M // 2
WIN = 8000
NWIN = HALF // WIN
VPW = WIN // 16
K = N_Q
KPAD = 5008
SEG = KPAD // 16
BUF = 5024


def _elementwise_body(logits_ref, boxes_t_ref, count_ref, ts_ref,
                      probs_ref, raw_t_ref, scaled_t_ref, eseq_ref):
    probs_ref[...] = jax.nn.sigmoid(logits_ref[...])

    c = boxes_t_ref[0, 0, :]
    l = boxes_t_ref[0, 1, :]
    lo = c - 0.5 * l
    hi = c + 0.5 * l
    raw_t_ref[0, 0, :] = lo
    raw_t_ref[0, 1, :] = hi
    ts = ts_ref[0, 0, 0]
    scaled_t_ref[0, 0, :] = jnp.clip(lo, 0.0, 1.0) * ts
    scaled_t_ref[0, 1, :] = jnp.clip(hi, 0.0, 1.0) * ts

    cnt = count_ref[0, 0, :]
    m = jnp.max(cnt)
    iota = lax.broadcasted_iota(jnp.int32, cnt.shape, 0)
    first_max = jnp.min(jnp.where(cnt == m, iota, jnp.int32(10**9)))
    eseq_ref[...] = jnp.maximum(first_max, 1).reshape(1, 1, 1)


def _sc_topk_body(probs_hbm, sbox_hbm,
                  scores_hbm, labels_hbm, qidx_hbm, tidx_hbm, boxes_hbm,
                  wbuf0, wbuf1, hist, red_own, red_par,
                  sval, sidxv, keyA, idxA, keyB, idxB, tmp1k, tmp1i,
                  offs, sbox, obox, olab, oqv, osc,
                  shist, sstat, skeys, sidxs):
    c = lax.axis_index("c")
    s = lax.axis_index("s")
    b = c * 8 + s // 2
    h = s % 2
    part = s + 1 - 2 * h
    base = b * M + h * HALF

    lane = lax.iota(jnp.int32, 16)
    ones = jnp.ones((16,), jnp.int32)
    zeros = jnp.zeros((16,), jnp.int32)

    def splat(x):
        return jnp.full((16,), x, jnp.int32)

    ksp = splat(K)
    psel = zeros
    a_gt = zeros
    w_gt = zeros
    dstar = zeros

    for level, shift in enumerate((20, 10, 0)):
        def zb(j, _):
            plsc.store_scatter(hist, [j * 16 + lane], zeros)
            return 0
        lax.fori_loop(0, 1024, zb, 0)

        psel_s = psel

        def wloop(w, _):
            pltpu.sync_copy(probs_hbm.at[pl.ds(base + w * WIN, WIN)], wbuf0)

            def hb(i, _):
                k = plsc.bitcast(plsc.load_gather(wbuf0, [i * 16 + lane]),
                                 jnp.int32)
                d = (k >> shift) & 1023
                if level == 0:
                    plsc.addupdate_scatter(hist, [d * 16 + lane], ones)
                else:
                    msk = (k >> (shift + 10)) == psel_s
                    plsc.addupdate_scatter(hist, [d * 16 + lane], ones,
                                           mask=msk)
                return 0
            return lax.fori_loop(0, VPW, hb, 0)
        lax.fori_loop(0, NWIN, wloop, 0)

        def rb(j, _):
            dvec = j * 16 + lane
            acc = zeros
            for l in range(16):
                acc = acc + plsc.load_gather(hist, [dvec * 16 + l])
            plsc.store_scatter(red_own, [dvec], acc)
            return 0
        lax.fori_loop(0, 64, rb, 0)

        pltpu.sync_copy(red_own, shist.at[s])
        plsc.subcore_barrier()
        pltpu.sync_copy(shist.at[part], red_par)

        def db(t, carry):
            r, rown, dst, found = carry
            d = splat(1023 - t)
            mo = plsc.load_gather(red_own, [d])
            mm = mo + plsc.load_gather(red_par, [d])
            hit = jnp.logical_and(jnp.logical_not(found),
                                  a_gt + r + mm >= ksp)
            dst = jnp.where(hit, d, dst)
            nf = jnp.logical_or(found, hit)
            r = r + jnp.where(nf, 0, mm)
            rown = rown + jnp.where(nf, 0, mo)
            return r, rown, dst, nf
        r, rown, dstar, _ = lax.fori_loop(
            0, 1024, db, (zeros, zeros, zeros, lane < 0))
        a_gt = a_gt + r
        w_gt = w_gt + rown
        psel = psel * 1024 + dstar
        plsc.subcore_barrier()

    thresh = psel
    eq_own = plsc.load_gather(red_own, [dstar])

    stat = jnp.where(lane == 0, w_gt, jnp.where(lane == 1, eq_own, zeros))
    plsc.store_scatter(red_own, [lane], stat)
    pltpu.sync_copy(red_own.at[pl.ds(0, 16)], sstat.at[s])
    plsc.subcore_barrier()
    pltpu.sync_copy(sstat.at[part], red_par.at[pl.ds(0, 16)])
    rp = red_par[pl.ds(0, 16)]
    gt_p = splat(rp[0])
    eq_p = splat(rp[1])

    gt0 = jnp.where(h == 0, w_gt, gt_p)
    gt1 = jnp.where(h == 0, gt_p, w_gt)
    eq0 = jnp.where(h == 0, eq_own, eq_p)
    need = ksp - gt0 - gt1
    quota0 = jnp.minimum(eq0, need)
    quota1 = need - quota0
    myquota = jnp.where(h == 0, quota0, quota1)
    n0 = gt0 + quota0

    tsp = thresh
    qsp = myquota
    def cwloop(w, carry):
        pltpu.sync_copy(probs_hbm.at[pl.ds(base + w * WIN, WIN)], wbuf0)
        gbase = splat(h * HALF) + w * WIN + lane

        def cb(i, carry):
            off, eqseen = carry
            k = plsc.bitcast(plsc.load_gather(wbuf0, [i * 16 + lane]),
                             jnp.int32)
            gt = k > tsp
            eq = k == tsp
            eqcum = plsc.cumsum(eq.astype(jnp.int32))
            sel = jnp.logical_or(gt, jnp.logical_and(
                eq, eqseen + eqcum <= qsp))
            mc = plsc.cumsum(sel.astype(jnp.int32))
            addr = off + mc - 1
            plsc.store_scatter(sval, [addr], k, mask=sel)
            plsc.store_scatter(sidxv, [addr], gbase + i * 16, mask=sel)
            off = off + plsc.all_reduce_population_count(sel)
            eqseen = eqseen + plsc.all_reduce_population_count(eq)
            return off, eqseen
        return lax.fori_loop(0, VPW, cb, carry)
    off, eqseen = lax.fori_loop(0, NWIN, cwloop, (zeros, zeros))

    pltpu.sync_copy(sval, skeys.at[s])
    pltpu.sync_copy(sidxv, sidxs.at[s])
    plsc.subcore_barrier()

    @pl.when(h == 0)
    def _sorter():
        pltpu.sync_copy(skeys.at[s], keyB)
        pltpu.sync_copy(sidxs.at[s], idxB)
        pltpu.sync_copy(skeys.at[s + 1], tmp1k)
        pltpu.sync_copy(sidxs.at[s + 1], tmp1i)
        pltpu.sync_copy(sbox_hbm.at[pl.ds(b * 2 * N_Q, 2 * N_Q)], sbox)

        n0s = n0

        def mb(i, _):
            j = i * 16 + lane
            inb = j < n0s
            j2 = jnp.maximum(j - n0s, 0)
            k0 = plsc.load_gather(keyB, [j])
            k1 = plsc.load_gather(tmp1k, [j2])
            v0 = plsc.load_gather(idxB, [j])
            v1 = plsc.load_gather(tmp1i, [j2])
            live = j < K
            k = jnp.where(jnp.logical_and(inb, live), k0,
                          jnp.where(live, k1, 0))
            v = jnp.where(jnp.logical_and(inb, live), v0,
                          jnp.where(live, v1, 0))
            plsc.store_scatter(keyA, [j], k)
            plsc.store_scatter(idxA, [j], v)
            return 0
        lax.fori_loop(0, SEG, mb, 0)

        bufs = [(keyA, idxA, keyB, idxB), (keyB, idxB, keyA, idxA)]
        for p in range(5):
            srck, srcv, dstk, dstv = bufs[p % 2]
            shift = 6 * p

            def zb2(j, _):
                plsc.store_scatter(hist, [j * 16 + lane], zeros)
                return 0
            lax.fori_loop(0, 64, zb2, 0)

            def hb2(i, _, srck=srck, shift=shift):
                k = plsc.load_gather(srck, [lane * SEG + i])
                d = 63 - ((k >> shift) & 63)
                plsc.addupdate_scatter(hist, [d * 16 + lane], ones)
                return 0
            lax.fori_loop(0, SEG, hb2, 0)

            def ob(d, run):
                hv = plsc.load_gather(hist, [d * 16 + lane])
                cum = plsc.cumsum(hv)
                plsc.store_scatter(offs, [d * 16 + lane], run + cum - hv)
                return run + splat(jnp.sum(hv))
            lax.fori_loop(0, 64, ob, zeros)

            def pb(i, _, srck=srck, srcv=srcv, dstk=dstk, dstv=dstv,
                   shift=shift):
                ad = lane * SEG + i
                k = plsc.load_gather(srck, [ad])
                v = plsc.load_gather(srcv, [ad])
                d = 63 - ((k >> shift) & 63)
                ha = d * 16 + lane
                o = plsc.load_gather(offs, [ha])
                plsc.store_scatter(dstk, [o], k)
                plsc.store_scatter(dstv, [o], v)
                plsc.addupdate_scatter(offs, [ha], ones)
                return 0
            lax.fori_loop(0, SEG, pb, 0)

        def xb(i, _):
            j = i * 16 + lane
            k = plsc.load_gather(keyB, [j])
            v = plsc.load_gather(idxB, [j])
            q = v // N_CLASS
            lb = v - q * N_CLASS
            qc = jnp.minimum(q, N_Q - 1)
            lo = plsc.load_gather(sbox, [qc])
            hi = plsc.load_gather(sbox, [qc + N_Q])
            plsc.store_scatter(osc, [j], plsc.bitcast(k, jnp.float32))
            plsc.store_scatter(olab, [j], lb)
            plsc.store_scatter(oqv, [j], q)
            plsc.store_scatter(obox, [2 * j], lo)
            plsc.store_scatter(obox, [2 * j + 1], hi)
            return 0
        lax.fori_loop(0, SEG, xb, 0)

        pltpu.sync_copy(osc.at[pl.ds(0, K)], scores_hbm.at[pl.ds(b * K, K)])
        pltpu.sync_copy(olab.at[pl.ds(0, K)], labels_hbm.at[pl.ds(b * K, K)])
        pltpu.sync_copy(oqv.at[pl.ds(0, K)], qidx_hbm.at[pl.ds(b * K, K)])
        pltpu.sync_copy(idxB.at[pl.ds(0, K)], tidx_hbm.at[pl.ds(b * K, K)])
        pltpu.sync_copy(obox.at[pl.ds(0, 2 * K)],
                        boxes_hbm.at[pl.ds(b * 2 * K, 2 * K)])


@functools.partial(
    pl.kernel,
    out_type=[
        jax.ShapeDtypeStruct((N * K,), jnp.float32),
        jax.ShapeDtypeStruct((N * K,), jnp.int32),
        jax.ShapeDtypeStruct((N * K,), jnp.int32),
        jax.ShapeDtypeStruct((N * K,), jnp.int32),
        jax.ShapeDtypeStruct((N * 2 * K,), jnp.float32),
    ],
    mesh=plsc.VectorSubcoreMesh(core_axis_name="c", subcore_axis_name="s", num_cores=2, num_subcores=16),
    compiler_params=pltpu.CompilerParams(needs_layout_passes=False),
    scratch_types=[
        pltpu.VMEM((WIN,), jnp.float32),
        pltpu.VMEM((WIN,), jnp.float32),
        pltpu.VMEM((16384,), jnp.int32),
        pltpu.VMEM((1024,), jnp.int32),
        pltpu.VMEM((1024,), jnp.int32),
        pltpu.VMEM((BUF,), jnp.int32),
        pltpu.VMEM((BUF,), jnp.int32),
        pltpu.VMEM((BUF,), jnp.int32),
        pltpu.VMEM((BUF,), jnp.int32),
        pltpu.VMEM((BUF,), jnp.int32),
        pltpu.VMEM((BUF,), jnp.int32),
        pltpu.VMEM((BUF,), jnp.int32),
        pltpu.VMEM((BUF,), jnp.int32),
        pltpu.VMEM((1024,), jnp.int32),
        pltpu.VMEM((2 * N_Q,), jnp.float32),
        pltpu.VMEM((2 * KPAD,), jnp.float32),
        pltpu.VMEM((KPAD,), jnp.int32),
        pltpu.VMEM((KPAD,), jnp.int32),
        pltpu.VMEM((KPAD,), jnp.float32),
        pltpu.VMEM_SHARED((16, 1024), jnp.int32),
        pltpu.VMEM_SHARED((16, 16), jnp.int32),
        pltpu.VMEM_SHARED((16, BUF), jnp.int32),
        pltpu.VMEM_SHARED((16, BUF), jnp.int32),
    ],
)
def _sc_topk(*refs):
    _sc_topk_body(*refs)


def kernel(pred_logits, pred_boxes, pred_count, target_sizes):
    n, n_q, n_class = pred_logits.shape
    R = M // 128
    logits3d = pred_logits.reshape(n, R, 128)
    boxes_t = pred_boxes.transpose(0, 2, 1)
    count3 = pred_count.reshape(n, 1, pred_count.shape[1])
    ts3 = target_sizes.reshape(n, 1, 1)

    probs, raw_t, scaled_t, eseq3 = pl.pallas_call(
        _elementwise_body,
        grid=(n,),
        in_specs=[
            pl.BlockSpec((1, R, 128), lambda i: (i, 0, 0)),
            pl.BlockSpec((1, 2, n_q), lambda i: (i, 0, 0)),
            pl.BlockSpec((1, 1, pred_count.shape[1]), lambda i: (i, 0, 0)),
            pl.BlockSpec((1, 1, 1), lambda i: (i, 0, 0)),
        ],
        out_specs=[
            pl.BlockSpec((1, R, 128), lambda i: (i, 0, 0)),
            pl.BlockSpec((1, 2, n_q), lambda i: (i, 0, 0)),
            pl.BlockSpec((1, 2, n_q), lambda i: (i, 0, 0)),
            pl.BlockSpec((1, 1, 1), lambda i: (i, 0, 0)),
        ],
        out_shape=[
            jax.ShapeDtypeStruct((n, R, 128), jnp.float32),
            jax.ShapeDtypeStruct((n, 2, n_q), jnp.float32),
            jax.ShapeDtypeStruct((n, 2, n_q), jnp.float32),
            jax.ShapeDtypeStruct((n, 1, 1), jnp.int32),
        ],
    )(logits3d, boxes_t, count3, ts3)

    probs_flat = probs.reshape(n * M)
    scaled_flat = scaled_t.reshape(n * 2 * n_q)

    scores, labels, topk_boxes, topk_indexes, boxes_il = _sc_topk(
        probs_flat, scaled_flat)

    scores = scores.reshape(n, n_q)
    labels = labels.reshape(n, n_q)
    topk_boxes = topk_boxes.reshape(n, n_q)
    topk_indexes = topk_indexes.reshape(n, n_q)
    boxes = boxes_il.reshape(n, n_q, 2)
    raw_boxes = raw_t.transpose(0, 2, 1)
    eseq_lens = eseq3.reshape(n)
    return scores, labels, boxes, raw_boxes, topk_boxes, eseq_lens, topk_indexes


# speedup vs baseline: 6.5673x; 1.0107x over previous
import functools

import jax
import jax.numpy as jnp
from jax import lax
from jax.experimental import pallas as pl
from jax.experimental.pallas import tpu as pltpu
from jax.experimental.pallas import tpu_sc as plsc

N = 16
N_Q = 5000
N_CLASS = 80
M = N_Q * N_CLASS
HALF = M // 2
WIN = 8000
NWIN = HALF // WIN
VPW = WIN // 16
UNR = 1
K = N_Q
KPAD = 5008
SEG = KPAD // 16
SU = 1
BUF = 5024


def _elementwise_body(logits_ref, boxes_t_ref, count_ref, ts_ref,
                      probs_ref, raw_t_ref, scaled_t_ref, eseq_ref):
    probs_ref[...] = jax.nn.sigmoid(logits_ref[...])

    c = boxes_t_ref[0, 0, :]
    l = boxes_t_ref[0, 1, :]
    lo = c - 0.5 * l
    hi = c + 0.5 * l
    raw_t_ref[0, 0, :] = lo
    raw_t_ref[0, 1, :] = hi
    ts = ts_ref[0, 0, 0]
    scaled_t_ref[0, 0, :] = jnp.clip(lo, 0.0, 1.0) * ts
    scaled_t_ref[0, 1, :] = jnp.clip(hi, 0.0, 1.0) * ts

    cnt = count_ref[0, 0, :]
    m = jnp.max(cnt)
    iota = lax.broadcasted_iota(jnp.int32, cnt.shape, 0)
    first_max = jnp.min(jnp.where(cnt == m, iota, jnp.int32(10**9)))
    eseq_ref[...] = jnp.maximum(first_max, 1).reshape(1, 1, 1)


def _sc_topk_body(probs_hbm, sbox_hbm,
                  scores_hbm, labels_hbm, qidx_hbm, tidx_hbm, boxes_hbm,
                  wbuf0, wbuf1, hist, red_own, red_par,
                  sval, sidxv, keyA, idxA, keyB, idxB, tmp1k, tmp1i,
                  offs, sbox, obox, olab, oqv, osc, sem0, sem1,
                  shist, sstat, skeys, sidxs):
    c = lax.axis_index("c")
    s = lax.axis_index("s")
    b = c * 8 + s // 2
    h = s % 2
    part = s + 1 - 2 * h
    base = b * M + h * HALF

    lane = lax.iota(jnp.int32, 16)
    ones = jnp.ones((16,), jnp.int32)
    zeros = jnp.zeros((16,), jnp.int32)

    def splat(x):
        return jnp.full((16,), x, jnp.int32)

    def win_src(w):
        return probs_hbm.at[pl.ds(base + w * WIN, WIN)]

    def windowed_scan(process, carry):
        def wloop(w, carry):
            pltpu.sync_copy(win_src(w), wbuf0)
            return process(wbuf0, w, carry)
        return lax.fori_loop(0, NWIN, wloop, carry)

    ksp = splat(K)
    psel = zeros
    a_gt = zeros
    w_gt = zeros
    dstar = zeros

    for level, shift in enumerate((20, 10, 0)):
        def zb(j, _):
            for u in range(8):
                plsc.store_scatter(hist, [(j * 8 + u) * 16 + lane], zeros)
            return 0
        lax.fori_loop(0, 128, zb, 0)

        psel_s = psel

        def hproc(buf, w, carry, shift=shift, level=level, psel_s=psel_s):
            def hb(i, _):
                for u in range(UNR):
                    k = plsc.bitcast(
                        plsc.load_gather(buf, [(i * UNR + u) * 16 + lane]),
                        jnp.int32)
                    d = (k >> shift) & 1023
                    if level == 0:
                        plsc.addupdate_scatter(hist, [d * 16 + lane], ones)
                    else:
                        msk = (k >> (shift + 10)) == psel_s
                        plsc.addupdate_scatter(hist, [d * 16 + lane], ones,
                                               mask=msk)
                return 0
            return lax.fori_loop(0, VPW // UNR, hb, carry)
        windowed_scan(hproc, 0)

        def rb(j, _):
            dvec = j * 16 + lane
            acc = zeros
            for l in range(16):
                acc = acc + plsc.load_gather(hist, [dvec * 16 + l])
            plsc.store_scatter(red_own, [dvec], acc)
            return 0
        lax.fori_loop(0, 64, rb, 0)

        pltpu.sync_copy(red_own, shist.at[s])
        plsc.subcore_barrier()
        pltpu.sync_copy(shist.at[part], red_par)

        def db(t, carry):
            r, rown, dst, found = carry
            d = splat(1023 - t)
            mo = plsc.load_gather(red_own, [d])
            mm = mo + plsc.load_gather(red_par, [d])
            hit = jnp.logical_and(jnp.logical_not(found),
                                  a_gt + r + mm >= ksp)
            dst = jnp.where(hit, d, dst)
            nf = jnp.logical_or(found, hit)
            r = r + jnp.where(nf, 0, mm)
            rown = rown + jnp.where(nf, 0, mo)
            return r, rown, dst, nf
        r, rown, dstar, _ = lax.fori_loop(
            0, 1024, db, (zeros, zeros, zeros, lane < 0))
        a_gt = a_gt + r
        w_gt = w_gt + rown
        psel = psel * 1024 + dstar
        plsc.subcore_barrier()

    thresh = psel
    eq_own = plsc.load_gather(red_own, [dstar])

    stat = jnp.where(lane == 0, w_gt, jnp.where(lane == 1, eq_own, zeros))
    plsc.store_scatter(red_own, [lane], stat)
    pltpu.sync_copy(red_own.at[pl.ds(0, 16)], sstat.at[s])
    plsc.subcore_barrier()
    pltpu.sync_copy(sstat.at[part], red_par.at[pl.ds(0, 16)])
    rp = red_par[pl.ds(0, 16)]
    gt_p = splat(rp[0])
    eq_p = splat(rp[1])

    gt0 = jnp.where(h == 0, w_gt, gt_p)
    gt1 = jnp.where(h == 0, gt_p, w_gt)
    eq0 = jnp.where(h == 0, eq_own, eq_p)
    need = ksp - gt0 - gt1
    quota0 = jnp.minimum(eq0, need)
    quota1 = need - quota0
    myquota = jnp.where(h == 0, quota0, quota1)
    n0 = gt0 + quota0

    tsp = thresh
    qsp = myquota

    def cproc(buf, w, carry):
        gb = splat(h * HALF) + w * WIN + lane

        def cb(i, carry):
            off, eqseen = carry
            for u in range(UNR):
                k = plsc.bitcast(
                    plsc.load_gather(buf, [(i * UNR + u) * 16 + lane]),
                    jnp.int32)
                gt = k > tsp
                eq = k == tsp
                eqcum = plsc.cumsum(eq.astype(jnp.int32))
                sel = jnp.logical_or(gt, jnp.logical_and(
                    eq, eqseen + eqcum <= qsp))
                mc = plsc.cumsum(sel.astype(jnp.int32))
                addr = off + mc - 1
                plsc.store_scatter(sval, [addr], k, mask=sel)
                plsc.store_scatter(sidxv, [addr], gb + (i * UNR + u) * 16,
                                   mask=sel)
                off = off + plsc.all_reduce_population_count(sel)
                eqseen = eqseen + plsc.all_reduce_population_count(eq)
            return off, eqseen
        return lax.fori_loop(0, VPW // UNR, cb, carry)
    windowed_scan(cproc, (zeros, zeros))

    pltpu.sync_copy(sval, skeys.at[s])
    pltpu.sync_copy(sidxv, sidxs.at[s])
    plsc.subcore_barrier()

    @pl.when(h == 0)
    def _sorter():
        pltpu.sync_copy(skeys.at[s], keyB)
        pltpu.sync_copy(sidxs.at[s], idxB)
        pltpu.sync_copy(skeys.at[s + 1], tmp1k)
        pltpu.sync_copy(sidxs.at[s + 1], tmp1i)
        pltpu.sync_copy(sbox_hbm.at[pl.ds(b * 2 * N_Q, 2 * N_Q)], sbox)

        n0s = n0

        def mb(i, _):
            for u in range(SU):
                j = (i * SU + u) * 16 + lane
                inb = j < n0s
                j2 = jnp.maximum(j - n0s, 0)
                k0 = plsc.load_gather(keyB, [j])
                k1 = plsc.load_gather(tmp1k, [j2])
                v0 = plsc.load_gather(idxB, [j])
                v1 = plsc.load_gather(tmp1i, [j2])
                live = j < K
                k = jnp.where(jnp.logical_and(inb, live), k0,
                              jnp.where(live, k1, 0))
                v = jnp.where(jnp.logical_and(inb, live), v0,
                              jnp.where(live, v1, 0))
                plsc.store_scatter(keyA, [j], k)
                plsc.store_scatter(idxA, [j], v)
            return 0
        lax.fori_loop(0, SEG // SU, mb, 0)

        bufs = [(keyA, idxA, keyB, idxB), (keyB, idxB, keyA, idxA)]
        for p in range(5):
            srck, srcv, dstk, dstv = bufs[p % 2]
            shift = 6 * p

            def zb2(j, _):
                plsc.store_scatter(hist, [j * 16 + lane], zeros)
                return 0
            lax.fori_loop(0, 64, zb2, 0)

            def hb2(i, _, srck=srck, shift=shift):
                for u in range(SU):
                    k = plsc.load_gather(srck, [lane * SEG + i * SU + u])
                    d = 63 - ((k >> shift) & 63)
                    plsc.addupdate_scatter(hist, [d * 16 + lane], ones)
                return 0
            lax.fori_loop(0, SEG // SU, hb2, 0)

            def ob(d, run):
                hv = plsc.load_gather(hist, [d * 16 + lane])
                cum = plsc.cumsum(hv)
                plsc.store_scatter(offs, [d * 16 + lane], run + cum - hv)
                return run + splat(jnp.sum(hv))
            lax.fori_loop(0, 64, ob, zeros)

            def pb(i, _, srck=srck, srcv=srcv, dstk=dstk, dstv=dstv,
                   shift=shift):
                for u in range(SU):
                    ad = lane * SEG + i * SU + u
                    k = plsc.load_gather(srck, [ad])
                    v = plsc.load_gather(srcv, [ad])
                    d = 63 - ((k >> shift) & 63)
                    ha = d * 16 + lane
                    o = plsc.load_gather(offs, [ha])
                    plsc.store_scatter(dstk, [o], k)
                    plsc.store_scatter(dstv, [o], v)
                    plsc.addupdate_scatter(offs, [ha], ones)
                return 0
            lax.fori_loop(0, SEG // SU, pb, 0)

        def xb(i, _):
            for u in range(SU):
                j = (i * SU + u) * 16 + lane
                k = plsc.load_gather(keyB, [j])
                v = plsc.load_gather(idxB, [j])
                q = v // N_CLASS
                lb = v - q * N_CLASS
                qc = jnp.minimum(q, N_Q - 1)
                lo = plsc.load_gather(sbox, [qc])
                hi = plsc.load_gather(sbox, [qc + N_Q])
                plsc.store_scatter(osc, [j], plsc.bitcast(k, jnp.float32))
                plsc.store_scatter(olab, [j], lb)
                plsc.store_scatter(oqv, [j], q)
                plsc.store_scatter(obox, [2 * j], lo)
                plsc.store_scatter(obox, [2 * j + 1], hi)
            return 0
        lax.fori_loop(0, SEG // SU, xb, 0)

        pltpu.sync_copy(osc.at[pl.ds(0, K)], scores_hbm.at[pl.ds(b * K, K)])
        pltpu.sync_copy(olab.at[pl.ds(0, K)], labels_hbm.at[pl.ds(b * K, K)])
        pltpu.sync_copy(oqv.at[pl.ds(0, K)], qidx_hbm.at[pl.ds(b * K, K)])
        pltpu.sync_copy(idxB.at[pl.ds(0, K)], tidx_hbm.at[pl.ds(b * K, K)])
        pltpu.sync_copy(obox.at[pl.ds(0, 2 * K)],
                        boxes_hbm.at[pl.ds(b * 2 * K, 2 * K)])


@functools.partial(
    pl.kernel,
    out_type=[
        jax.ShapeDtypeStruct((N * K,), jnp.float32),
        jax.ShapeDtypeStruct((N * K,), jnp.int32),
        jax.ShapeDtypeStruct((N * K,), jnp.int32),
        jax.ShapeDtypeStruct((N * K,), jnp.int32),
        jax.ShapeDtypeStruct((N * 2 * K,), jnp.float32),
    ],
    mesh=plsc.VectorSubcoreMesh(core_axis_name="c", subcore_axis_name="s", num_cores=2, num_subcores=16),
    compiler_params=pltpu.CompilerParams(needs_layout_passes=False),
    scratch_types=[
        pltpu.VMEM((WIN,), jnp.float32),
        pltpu.VMEM((WIN,), jnp.float32),
        pltpu.VMEM((16384,), jnp.int32),
        pltpu.VMEM((1024,), jnp.int32),
        pltpu.VMEM((1024,), jnp.int32),
        pltpu.VMEM((BUF,), jnp.int32),
        pltpu.VMEM((BUF,), jnp.int32),
        pltpu.VMEM((BUF,), jnp.int32),
        pltpu.VMEM((BUF,), jnp.int32),
        pltpu.VMEM((BUF,), jnp.int32),
        pltpu.VMEM((BUF,), jnp.int32),
        pltpu.VMEM((BUF,), jnp.int32),
        pltpu.VMEM((BUF,), jnp.int32),
        pltpu.VMEM((1024,), jnp.int32),
        pltpu.VMEM((2 * N_Q,), jnp.float32),
        pltpu.VMEM((2 * KPAD,), jnp.float32),
        pltpu.VMEM((KPAD,), jnp.int32),
        pltpu.VMEM((KPAD,), jnp.int32),
        pltpu.VMEM((KPAD,), jnp.float32),
        pltpu.SemaphoreType.DMA,
        pltpu.SemaphoreType.DMA,
        pltpu.VMEM_SHARED((16, 1024), jnp.int32),
        pltpu.VMEM_SHARED((16, 16), jnp.int32),
        pltpu.VMEM_SHARED((16, BUF), jnp.int32),
        pltpu.VMEM_SHARED((16, BUF), jnp.int32),
    ],
)
def _sc_topk(*refs):
    _sc_topk_body(*refs)


def kernel(pred_logits, pred_boxes, pred_count, target_sizes):
    n, n_q, n_class = pred_logits.shape
    R = M // 128
    logits3d = pred_logits.reshape(n, R, 128)
    boxes_t = pred_boxes.transpose(0, 2, 1)
    count3 = pred_count.reshape(n, 1, pred_count.shape[1])
    ts3 = target_sizes.reshape(n, 1, 1)

    probs, raw_t, scaled_t, eseq3 = pl.pallas_call(
        _elementwise_body,
        grid=(n,),
        in_specs=[
            pl.BlockSpec((1, R, 128), lambda i: (i, 0, 0)),
            pl.BlockSpec((1, 2, n_q), lambda i: (i, 0, 0)),
            pl.BlockSpec((1, 1, pred_count.shape[1]), lambda i: (i, 0, 0)),
            pl.BlockSpec((1, 1, 1), lambda i: (i, 0, 0)),
        ],
        out_specs=[
            pl.BlockSpec((1, R, 128), lambda i: (i, 0, 0)),
            pl.BlockSpec((1, 2, n_q), lambda i: (i, 0, 0)),
            pl.BlockSpec((1, 2, n_q), lambda i: (i, 0, 0)),
            pl.BlockSpec((1, 1, 1), lambda i: (i, 0, 0)),
        ],
        out_shape=[
            jax.ShapeDtypeStruct((n, R, 128), jnp.float32),
            jax.ShapeDtypeStruct((n, 2, n_q), jnp.float32),
            jax.ShapeDtypeStruct((n, 2, n_q), jnp.float32),
            jax.ShapeDtypeStruct((n, 1, 1), jnp.int32),
        ],
    )(logits3d, boxes_t, count3, ts3)

    probs_flat = probs.reshape(n * M)
    scaled_flat = scaled_t.reshape(n * 2 * n_q)

    scores, labels, topk_boxes, topk_indexes, boxes_il = _sc_topk(
        probs_flat, scaled_flat)

    scores = scores.reshape(n, n_q)
    labels = labels.reshape(n, n_q)
    topk_boxes = topk_boxes.reshape(n, n_q)
    topk_indexes = topk_indexes.reshape(n, n_q)
    boxes = boxes_il.reshape(n, n_q, 2)
    raw_boxes = raw_t.transpose(0, 2, 1)
    eseq_lens = eseq3.reshape(n)
    return scores, labels, boxes, raw_boxes, topk_boxes, eseq_lens, topk_indexes
